# Initial kernel scaffold; baseline (speedup 1.0000x reference)
import jax
import jax.numpy as jnp
from jax.experimental import pallas as pl


def kernel(inputs):
    raise NotImplementedError("write your pallas kernel here")


# speedup vs baseline: 5.4738x; 5.4738x over previous
import functools

import jax
import jax.numpy as jnp
import numpy as np
from jax import lax
from jax.experimental import pallas as pl
from jax.experimental.pallas import tpu as pltpu
from jax.experimental.pallas import tpu_sc as plsc

B = 8
NCAT = 96 * 24 * 24
NTPR = 4
CHUNK = NCAT // NTPR
NGRP = CHUNK // 16
_BITS = [1 << k for k in range(15, -1, -1)]


def _threefry2x32(k1, k2, x0, x1):
    def rotl(x, d):
        return (x << np.uint32(d)) | (x >> np.uint32(32 - d))

    ks = [np.uint32(k1), np.uint32(k2),
          np.uint32(k1) ^ np.uint32(k2) ^ np.uint32(0x1BD11BDA)]
    rots = [np.array([13, 15, 26, 6], np.uint32),
            np.array([17, 29, 16, 24], np.uint32)]
    x0 = x0 + ks[0]
    x1 = x1 + ks[1]
    for i in range(5):
        for r in rots[i % 2]:
            x0 = x0 + x1
            x1 = rotl(x1, r)
            x1 = x0 ^ x1
        x0 = x0 + ks[(i + 1) % 3]
        x1 = x1 + ks[(i + 2) % 3] + np.uint32(i + 1)
    return x0, x1


def _sorted_uniforms(seed, b, n):
    zb = np.zeros(b, np.uint32)
    k1, k2 = _threefry2x32(np.uint32(0), np.uint32(seed), zb,
                           np.arange(b, dtype=np.uint32))
    rows = []
    zn = np.zeros(n, np.uint32)
    cn = np.arange(n, dtype=np.uint32)
    for i in range(b):
        b1, b2 = _threefry2x32(k1[i], k2[i], zn, cn)
        fb = ((b1 ^ b2) >> np.uint32(9)) | np.uint32(0x3F800000)
        rows.append(np.sort(fb.view(np.float32) - np.float32(1.0)))
    return np.stack(rows)


_U_SORTED = _sorted_uniforms(42, B, NCAT)


def _sap_body(x_hbm, cdf_hbm, u_hbm, s_hbm, out_hbm, u_v, q_v, f_v, x_v, s_v, c_v):
    wid = lax.axis_index("s") * 2 + lax.axis_index("c")
    row = wid // NTPR
    quarter = wid % NTPR
    start = quarter * CHUNK

    pltpu.sync_copy(u_hbm.at[row], u_v)
    pltpu.sync_copy(cdf_hbm.at[row, pl.ds(start, CHUNK)], q_v)
    pltpu.sync_copy(x_hbm.at[row, pl.ds(start, CHUNK)], x_v)
    pltpu.sync_copy(s_hbm.at[row], s_v)
    prev_start = jnp.where(quarter == 0, 0, start - 16)
    pltpu.sync_copy(cdf_hbm.at[row, pl.ds(prev_start, 16)], c_v)

    lane = lax.iota(jnp.int32, 16)

    def search16(q16):
        pos = jnp.zeros((16,), jnp.int32)
        for b in _BITS:
            cand = pos + b
            valid = cand <= NCAT
            gidx = jnp.minimum(cand, NCAT) - 1
            val = plsc.load_gather(u_v, [gidx])
            take = valid & (val <= q16)
            pos = jnp.where(take, cand, pos)
        return pos

    f_prev = search16(c_v[...])
    f_v[pl.ds(0, 16)] = jnp.where(quarter == 0, jnp.zeros((16,), jnp.int32), f_prev)

    def pass1(g, carry):
        q16 = q_v[pl.ds(g * 16, 16)]
        jvec = (start + g * 16) + lane
        q16 = jnp.where(jvec == NCAT - 1, jnp.float32(2.0), q16)
        f_v[pl.ds(16 + g * 16, 16)] = search16(q16)
        return carry

    lax.fori_loop(0, NGRP, pass1, 0)

    def pass2(g, carry):
        f_cur = f_v[pl.ds(16 + g * 16, 16)]
        f_pre = f_v[pl.ds(15 + g * 16, 16)]
        keep = f_cur > f_pre
        xv = x_v[pl.ds(g * 16, 16)]
        pv = jnp.abs(xv) / s_v[...]
        d = (1.0 - pv) - 1.0
        t = (-d) * (1.0 + d * (-0.5 + d * (jnp.float32(1.0 / 3.0) + d * -0.25)))
        powterm = jnp.exp(jnp.float32(-float(NCAT)) * t)
        denom = (1.0 - powterm) + jnp.float32(1e-12)
        x_v[pl.ds(g * 16, 16)] = jnp.where(keep, xv, jnp.float32(0.0)) / denom
        return carry

    lax.fori_loop(0, NGRP, pass2, 0)

    pltpu.sync_copy(x_v, out_hbm.at[row, pl.ds(start, CHUNK)])


_sap_call = functools.partial(
    pl.kernel,
    mesh=plsc.VectorSubcoreMesh(core_axis_name="c", subcore_axis_name="s"),
    out_type=jax.ShapeDtypeStruct((B, NCAT), jnp.float32),
    compiler_params=pltpu.CompilerParams(
        use_tc_tiling_on_sc=False, needs_layout_passes=False),
    scratch_types=[
        pltpu.VMEM((NCAT,), jnp.float32),
        pltpu.VMEM((CHUNK,), jnp.float32),
        pltpu.VMEM((CHUNK + 16,), jnp.int32),
        pltpu.VMEM((CHUNK,), jnp.float32),
        pltpu.VMEM((16,), jnp.float32),
        pltpu.VMEM((16,), jnp.float32),
    ],
)(_sap_body)


def kernel(inputs):
    b, c, h, w = inputs.shape
    ncat = c * h * w
    x2 = inputs.reshape(b, ncat)
    abs_in = jnp.abs(x2)
    s = jnp.sum(abs_in, axis=-1, keepdims=True)
    p = abs_in / s
    cdf = jnp.cumsum(p, axis=-1)
    u_sorted = jnp.asarray(_U_SORTED)
    s_b = jnp.broadcast_to(s, (b, 16))
    out = _sap_call(x2, cdf, u_sorted, s_b)
    return out.reshape(b, c, h, w)


# speedup vs baseline: 8.8518x; 1.6171x over previous
import functools

import jax
import jax.numpy as jnp
import numpy as np
from jax import lax
from jax.experimental import pallas as pl
from jax.experimental.pallas import tpu as pltpu
from jax.experimental.pallas import tpu_sc as plsc

B = 8
NCAT = 96 * 24 * 24
NTPR = 4
CHUNK = NCAT // NTPR
NGRP = CHUNK // 16
_BITS = [1 << k for k in range(15, -1, -1)]


def _threefry2x32(k1, k2, x0, x1):
    def rotl(x, d):
        return (x << np.uint32(d)) | (x >> np.uint32(32 - d))

    ks = [np.uint32(k1), np.uint32(k2),
          np.uint32(k1) ^ np.uint32(k2) ^ np.uint32(0x1BD11BDA)]
    rots = [np.array([13, 15, 26, 6], np.uint32),
            np.array([17, 29, 16, 24], np.uint32)]
    x0 = x0 + ks[0]
    x1 = x1 + ks[1]
    for i in range(5):
        for r in rots[i % 2]:
            x0 = x0 + x1
            x1 = rotl(x1, r)
            x1 = x0 ^ x1
        x0 = x0 + ks[(i + 1) % 3]
        x1 = x1 + ks[(i + 2) % 3] + np.uint32(i + 1)
    return x0, x1


def _sorted_uniforms(seed, b, n):
    zb = np.zeros(b, np.uint32)
    k1, k2 = _threefry2x32(np.uint32(0), np.uint32(seed), zb,
                           np.arange(b, dtype=np.uint32))
    rows = []
    zn = np.zeros(n, np.uint32)
    cn = np.arange(n, dtype=np.uint32)
    for i in range(b):
        b1, b2 = _threefry2x32(k1[i], k2[i], zn, cn)
        fb = ((b1 ^ b2) >> np.uint32(9)) | np.uint32(0x3F800000)
        rows.append(np.sort(fb.view(np.float32) - np.float32(1.0)))
    return np.stack(rows)


_U_SORTED = _sorted_uniforms(42, B, NCAT)


def _sap_body(x_hbm, cdf_hbm, u_hbm, s_hbm, out_hbm, u_v, q_v, f_v, x_v, s_v, c_v):
    wid = lax.axis_index("s") * 2 + lax.axis_index("c")
    row = wid // NTPR
    quarter = wid % NTPR
    start = quarter * CHUNK

    pltpu.sync_copy(u_hbm.at[row], u_v)
    pltpu.sync_copy(cdf_hbm.at[row, pl.ds(start, CHUNK)], q_v)
    pltpu.sync_copy(x_hbm.at[row, pl.ds(start, CHUNK)], x_v)
    pltpu.sync_copy(s_hbm.at[row], s_v)
    prev_start = jnp.where(quarter == 0, 0, start - 16)
    pltpu.sync_copy(cdf_hbm.at[row, pl.ds(prev_start, 16)], c_v)

    lane = lax.iota(jnp.int32, 16)

    def search16(q16):
        pos = jnp.zeros((16,), jnp.int32)
        for b in _BITS:
            cand = pos + b
            valid = cand <= NCAT
            gidx = jnp.minimum(cand, NCAT) - 1
            val = plsc.load_gather(u_v, [gidx])
            take = valid & (val <= q16)
            pos = jnp.where(take, cand, pos)
        return pos

    f_prev = search16(c_v[...])
    f_v[pl.ds(0, 16)] = jnp.where(quarter == 0, jnp.zeros((16,), jnp.int32), f_prev)

    @plsc.parallel_loop(0, NGRP, 1, unroll=8)
    def pass1(g):
        q16 = q_v[pl.ds(g * 16, 16)]
        jvec = (start + g * 16) + lane
        q16 = jnp.where(jvec == NCAT - 1, jnp.float32(2.0), q16)
        f_v[pl.ds(16 + g * 16, 16)] = search16(q16)

    @plsc.parallel_loop(0, NGRP, 1, unroll=4)
    def pass2(g):
        f_cur = f_v[pl.ds(16 + g * 16, 16)]
        f_pre = f_v[pl.ds(15 + g * 16, 16)]
        keep = f_cur > f_pre
        xv = x_v[pl.ds(g * 16, 16)]
        pv = jnp.abs(xv) / s_v[...]
        d = (1.0 - pv) - 1.0
        t = (-d) * (1.0 + d * (-0.5 + d * (jnp.float32(1.0 / 3.0) + d * -0.25)))
        powterm = jnp.exp(jnp.float32(-float(NCAT)) * t)
        denom = (1.0 - powterm) + jnp.float32(1e-12)
        x_v[pl.ds(g * 16, 16)] = jnp.where(keep, xv, jnp.float32(0.0)) / denom

    pltpu.sync_copy(x_v, out_hbm.at[row, pl.ds(start, CHUNK)])


_sap_call = functools.partial(
    pl.kernel,
    mesh=plsc.VectorSubcoreMesh(core_axis_name="c", subcore_axis_name="s"),
    out_type=jax.ShapeDtypeStruct((B, NCAT), jnp.float32),
    compiler_params=pltpu.CompilerParams(
        use_tc_tiling_on_sc=False, needs_layout_passes=False),
    scratch_types=[
        pltpu.VMEM((NCAT,), jnp.float32),
        pltpu.VMEM((CHUNK,), jnp.float32),
        pltpu.VMEM((CHUNK + 16,), jnp.int32),
        pltpu.VMEM((CHUNK,), jnp.float32),
        pltpu.VMEM((16,), jnp.float32),
        pltpu.VMEM((16,), jnp.float32),
    ],
)(_sap_body)


def kernel(inputs):
    b, c, h, w = inputs.shape
    ncat = c * h * w
    x2 = inputs.reshape(b, ncat)
    abs_in = jnp.abs(x2)
    s = jnp.sum(abs_in, axis=-1, keepdims=True)
    p = abs_in / s
    cdf = jnp.cumsum(p, axis=-1)
    u_sorted = jnp.asarray(_U_SORTED)
    s_b = jnp.broadcast_to(s, (b, 16))
    out = _sap_call(x2, cdf, u_sorted, s_b)
    return out.reshape(b, c, h, w)


# speedup vs baseline: 14.6258x; 1.6523x over previous
import functools

import jax
import jax.numpy as jnp
import numpy as np
from jax import lax
from jax.experimental import pallas as pl
from jax.experimental.pallas import tpu as pltpu
from jax.experimental.pallas import tpu_sc as plsc

B = 8
NCAT = 96 * 24 * 24
NTPR = 4
CHUNK = NCAT // NTPR
NGRP = CHUNK // 16
BLK = 128
NBLK_T = CHUNK // BLK
NBLK_R = NCAT // BLK
PADB = 112
_BITS = [1 << k for k in range(15, -1, -1)]


def _threefry2x32(k1, k2, x0, x1):
    def rotl(x, d):
        return (x << np.uint32(d)) | (x >> np.uint32(32 - d))

    ks = [np.uint32(k1), np.uint32(k2),
          np.uint32(k1) ^ np.uint32(k2) ^ np.uint32(0x1BD11BDA)]
    rots = [np.array([13, 15, 26, 6], np.uint32),
            np.array([17, 29, 16, 24], np.uint32)]
    x0 = x0 + ks[0]
    x1 = x1 + ks[1]
    for i in range(5):
        for r in rots[i % 2]:
            x0 = x0 + x1
            x1 = rotl(x1, r)
            x1 = x0 ^ x1
        x0 = x0 + ks[(i + 1) % 3]
        x1 = x1 + ks[(i + 2) % 3] + np.uint32(i + 1)
    return x0, x1


def _sorted_uniforms(seed, b, n):
    zb = np.zeros(b, np.uint32)
    k1, k2 = _threefry2x32(np.uint32(0), np.uint32(seed), zb,
                           np.arange(b, dtype=np.uint32))
    rows = []
    zn = np.zeros(n, np.uint32)
    cn = np.arange(n, dtype=np.uint32)
    for i in range(b):
        b1, b2 = _threefry2x32(k1[i], k2[i], zn, cn)
        fb = ((b1 ^ b2) >> np.uint32(9)) | np.uint32(0x3F800000)
        rows.append(np.sort(fb.view(np.float32) - np.float32(1.0)))
    return np.stack(rows)


_U_SORTED = _sorted_uniforms(42, B, NCAT)


def _sap_body(x_hbm, p_hbm, u_hbm, out_hbm,
              u_v, p_v, x_v, w_v, f_v, c_v,
              bsl_v, stage_v, bsw_v, iv_v, gt_v,
              sh_bs, sh_tail):
    cid = lax.axis_index("c")
    sid = lax.axis_index("s")
    wid = cid * 16 + sid
    row = wid // NTPR
    row_l = sid // NTPR
    quarter = wid % NTPR
    start = quarter * CHUNK
    blk0 = quarter * NBLK_T

    pltpu.sync_copy(u_hbm.at[row], u_v)
    pltpu.sync_copy(p_hbm.at[row, pl.ds(start, CHUNK)], p_v)
    pltpu.sync_copy(x_hbm.at[row, pl.ds(start, CHUNK)], x_v)

    lane = lax.iota(jnp.int32, 16)

    ngroups = (NBLK_T + 15) // 16
    cblk = []
    gvalid = []
    for t in range(ngroups):
        lb = t * 16 + lane
        gvalid.append(lb < NBLK_T)
        cblk.append(jnp.minimum(lb, NBLK_T - 1) * BLK)

    def within_step(l, accs):
        new = []
        for t in range(ngroups):
            idx = cblk[t] + l
            val = plsc.load_gather(p_v, [idx])
            acc = accs[t] + val
            plsc.store_scatter(w_v, [idx], acc, mask=gvalid[t])
            new.append(acc)
        return tuple(new)

    accs = lax.fori_loop(0, BLK, within_step,
                         tuple(jnp.zeros((16,), jnp.float32)
                               for _ in range(ngroups)))
    for t in range(ngroups):
        bsl_v[pl.ds(t * 16, 16)] = accs[t]

    pltpu.sync_copy(bsl_v, sh_bs.at[row_l, pl.ds(quarter * PADB, PADB)])
    plsc.subcore_barrier()
    pltpu.sync_copy(sh_bs.at[row_l], stage_v)

    glc = jnp.minimum(lane, 3)

    def bsw_step(m, acc2):
        blk = glc * BLK + m
        validg = (lane < 4) & (blk < NBLK_R)
        blkc = jnp.minimum(blk, NBLK_R - 1)
        sidx = (blkc // NBLK_T) * PADB + blkc % NBLK_T
        val = plsc.load_gather(stage_v, [sidx])
        acc2 = acc2 + jnp.where(validg, val, jnp.float32(0.0))
        plsc.store_scatter(bsw_v, [glc * BLK + m], acc2, mask=lane < 4)
        return acc2

    acc2 = lax.fori_loop(0, BLK, bsw_step, jnp.zeros((16,), jnp.float32))
    gt_v[...] = acc2

    g0 = plsc.load_gather(gt_v, [jnp.zeros((16,), jnp.int32)])
    g1 = plsc.load_gather(gt_v, [jnp.full((16,), 1, jnp.int32)])
    g2 = plsc.load_gather(gt_v, [jnp.full((16,), 2, jnp.int32)])
    goff = [jnp.zeros((16,), jnp.float32), g0, g0 + g1, (g0 + g1) + g2]
    for i in range(NBLK_R // 16):
        g = (i * 16) // BLK
        iv_v[pl.ds(i * 16, 16)] = bsw_v[pl.ds(i * 16, 16)] + goff[g]

    @plsc.parallel_loop(0, NBLK_T, 1, unroll=4)
    def cdf_build(l):
        gblk = blk0 + l
        pidx = jnp.full((16,), 0, jnp.int32) + jnp.maximum(gblk - 1, 0)
        offv = plsc.load_gather(iv_v, [pidx])
        offv = jnp.where(gblk == 0, jnp.float32(0.0), offv)
        for m in range(BLK // 16):
            k = l * BLK + m * 16
            w_v[pl.ds(k, 16)] = w_v[pl.ds(k, 16)] + offv

    pltpu.sync_copy(w_v.at[pl.ds(CHUNK - 16, 16)], sh_tail.at[row_l, quarter])
    plsc.subcore_barrier()
    pltpu.sync_copy(sh_tail.at[row_l, jnp.maximum(quarter - 1, 0)], c_v)

    def search16(q16):
        pos = jnp.zeros((16,), jnp.int32)
        for b in _BITS:
            cand = pos + b
            valid = cand <= NCAT
            gidx = jnp.minimum(cand, NCAT) - 1
            val = plsc.load_gather(u_v, [gidx])
            take = valid & (val <= q16)
            pos = jnp.where(take, cand, pos)
        return pos

    f_prev = search16(c_v[...])
    f_v[pl.ds(0, 16)] = jnp.where(quarter == 0, jnp.zeros((16,), jnp.int32), f_prev)

    @plsc.parallel_loop(0, NGRP, 1, unroll=8)
    def pass1(g):
        q16 = w_v[pl.ds(g * 16, 16)]
        jvec = (start + g * 16) + lane
        q16 = jnp.where(jvec == NCAT - 1, jnp.float32(2.0), q16)
        f_v[pl.ds(16 + g * 16, 16)] = search16(q16)

    @plsc.parallel_loop(0, NGRP, 1, unroll=4)
    def pass2(g):
        f_cur = f_v[pl.ds(16 + g * 16, 16)]
        f_pre = f_v[pl.ds(15 + g * 16, 16)]
        keep = f_cur > f_pre
        xv = x_v[pl.ds(g * 16, 16)]
        pv = p_v[pl.ds(g * 16, 16)]
        d = (1.0 - pv) - 1.0
        t = (-d) * (1.0 + d * (-0.5 + d * (jnp.float32(1.0 / 3.0) + d * -0.25)))
        powterm = jnp.exp(jnp.float32(-float(NCAT)) * t)
        denom = (1.0 - powterm) + jnp.float32(1e-12)
        x_v[pl.ds(g * 16, 16)] = jnp.where(keep, xv, jnp.float32(0.0)) / denom

    pltpu.sync_copy(x_v, out_hbm.at[row, pl.ds(start, CHUNK)])


_sap_call = functools.partial(
    pl.kernel,
    mesh=plsc.VectorSubcoreMesh(core_axis_name="c", subcore_axis_name="s"),
    out_type=jax.ShapeDtypeStruct((B, NCAT), jnp.float32),
    compiler_params=pltpu.CompilerParams(
        use_tc_tiling_on_sc=False, needs_layout_passes=False),
    scratch_types=[
        pltpu.VMEM((NCAT,), jnp.float32),
        pltpu.VMEM((CHUNK,), jnp.float32),
        pltpu.VMEM((CHUNK,), jnp.float32),
        pltpu.VMEM((CHUNK,), jnp.float32),
        pltpu.VMEM((CHUNK + 16,), jnp.int32),
        pltpu.VMEM((16,), jnp.float32),
        pltpu.VMEM((PADB,), jnp.float32),
        pltpu.VMEM((NTPR * PADB,), jnp.float32),
        pltpu.VMEM((4 * BLK,), jnp.float32),
        pltpu.VMEM((4 * BLK,), jnp.float32),
        pltpu.VMEM((16,), jnp.float32),
        pltpu.VMEM_SHARED((4, NTPR * PADB), jnp.float32),
        pltpu.VMEM_SHARED((4, NTPR, 16), jnp.float32),
    ],
)(_sap_body)


def kernel(inputs):
    b, c, h, w = inputs.shape
    ncat = c * h * w
    x2 = inputs.reshape(b, ncat)
    abs_in = jnp.abs(x2)
    s = jnp.sum(abs_in, axis=-1, keepdims=True)
    p = abs_in / s
    u_sorted = jnp.asarray(_U_SORTED)
    out = _sap_call(x2, p, u_sorted)
    return out.reshape(b, c, h, w)


# speedup vs baseline: 15.7689x; 1.0782x over previous
import functools

import jax
import jax.numpy as jnp
import numpy as np
from jax import lax
from jax.experimental import pallas as pl
from jax.experimental.pallas import tpu as pltpu
from jax.experimental.pallas import tpu_sc as plsc

B = 8
NCAT = 96 * 24 * 24
NTPR = 4
CHUNK = NCAT // NTPR
NGRP = CHUNK // 16
BLK = 128
NBLK_T = CHUNK // BLK
NBLK_R = NCAT // BLK
PADB = 112
_BITS = [1 << k for k in range(15, -1, -1)]


def _threefry2x32(k1, k2, x0, x1):
    def rotl(x, d):
        return (x << np.uint32(d)) | (x >> np.uint32(32 - d))

    ks = [np.uint32(k1), np.uint32(k2),
          np.uint32(k1) ^ np.uint32(k2) ^ np.uint32(0x1BD11BDA)]
    rots = [np.array([13, 15, 26, 6], np.uint32),
            np.array([17, 29, 16, 24], np.uint32)]
    x0 = x0 + ks[0]
    x1 = x1 + ks[1]
    for i in range(5):
        for r in rots[i % 2]:
            x0 = x0 + x1
            x1 = rotl(x1, r)
            x1 = x0 ^ x1
        x0 = x0 + ks[(i + 1) % 3]
        x1 = x1 + ks[(i + 2) % 3] + np.uint32(i + 1)
    return x0, x1


def _sorted_uniforms(seed, b, n):
    zb = np.zeros(b, np.uint32)
    k1, k2 = _threefry2x32(np.uint32(0), np.uint32(seed), zb,
                           np.arange(b, dtype=np.uint32))
    rows = []
    zn = np.zeros(n, np.uint32)
    cn = np.arange(n, dtype=np.uint32)
    for i in range(b):
        b1, b2 = _threefry2x32(k1[i], k2[i], zn, cn)
        fb = ((b1 ^ b2) >> np.uint32(9)) | np.uint32(0x3F800000)
        rows.append(np.sort(fb.view(np.float32) - np.float32(1.0)))
    return np.stack(rows)


_U_SORTED = _sorted_uniforms(42, B, NCAT)


def _sap_body(p_hbm, u_hbm, out_hbm,
              u_v, p_v, w_v, f_v, c_v,
              bsl_v, stage_v, bsw_v, iv_v, gt_v,
              sh_bs, sh_tail):
    cid = lax.axis_index("c")
    sid = lax.axis_index("s")
    wid = cid * 16 + sid
    row = wid // NTPR
    row_l = sid // NTPR
    quarter = wid % NTPR
    start = quarter * CHUNK
    blk0 = quarter * NBLK_T

    pltpu.sync_copy(u_hbm.at[row], u_v)
    pltpu.sync_copy(p_hbm.at[row, pl.ds(start, CHUNK)], p_v)

    lane = lax.iota(jnp.int32, 16)

    ngroups = (NBLK_T + 15) // 16
    cblk = []
    gvalid = []
    for t in range(ngroups):
        lb = t * 16 + lane
        gvalid.append(lb < NBLK_T)
        cblk.append(jnp.minimum(lb, NBLK_T - 1) * BLK)

    def within_step(l, accs):
        new = []
        for t in range(ngroups):
            idx = cblk[t] + l
            val = plsc.load_gather(p_v, [idx])
            acc = accs[t] + val
            plsc.store_scatter(w_v, [idx], acc, mask=gvalid[t])
            new.append(acc)
        return tuple(new)

    accs = lax.fori_loop(0, BLK, within_step,
                         tuple(jnp.zeros((16,), jnp.float32)
                               for _ in range(ngroups)))
    for t in range(ngroups):
        bsl_v[pl.ds(t * 16, 16)] = accs[t]

    pltpu.sync_copy(bsl_v, sh_bs.at[row_l, pl.ds(quarter * PADB, PADB)])
    plsc.subcore_barrier()
    pltpu.sync_copy(sh_bs.at[row_l], stage_v)

    glc = jnp.minimum(lane, 3)

    def bsw_step(m, acc2):
        blk = glc * BLK + m
        validg = (lane < 4) & (blk < NBLK_R)
        blkc = jnp.minimum(blk, NBLK_R - 1)
        sidx = (blkc // NBLK_T) * PADB + blkc % NBLK_T
        val = plsc.load_gather(stage_v, [sidx])
        acc2 = acc2 + jnp.where(validg, val, jnp.float32(0.0))
        plsc.store_scatter(bsw_v, [glc * BLK + m], acc2, mask=lane < 4)
        return acc2

    acc2 = lax.fori_loop(0, BLK, bsw_step, jnp.zeros((16,), jnp.float32))
    gt_v[...] = acc2

    g0 = plsc.load_gather(gt_v, [jnp.zeros((16,), jnp.int32)])
    g1 = plsc.load_gather(gt_v, [jnp.full((16,), 1, jnp.int32)])
    g2 = plsc.load_gather(gt_v, [jnp.full((16,), 2, jnp.int32)])
    goff = [jnp.zeros((16,), jnp.float32), g0, g0 + g1, (g0 + g1) + g2]
    for i in range(NBLK_R // 16):
        g = (i * 16) // BLK
        iv_v[pl.ds(i * 16, 16)] = bsw_v[pl.ds(i * 16, 16)] + goff[g]

    @plsc.parallel_loop(0, NBLK_T, 1, unroll=4)
    def cdf_build(l):
        gblk = blk0 + l
        pidx = jnp.full((16,), 0, jnp.int32) + jnp.maximum(gblk - 1, 0)
        offv = plsc.load_gather(iv_v, [pidx])
        offv = jnp.where(gblk == 0, jnp.float32(0.0), offv)
        for m in range(BLK // 16):
            k = l * BLK + m * 16
            w_v[pl.ds(k, 16)] = w_v[pl.ds(k, 16)] + offv

    pltpu.sync_copy(w_v.at[pl.ds(CHUNK - 16, 16)], sh_tail.at[row_l, quarter])
    plsc.subcore_barrier()
    pltpu.sync_copy(sh_tail.at[row_l, jnp.maximum(quarter - 1, 0)], c_v)

    def search16(q16):
        pos = jnp.zeros((16,), jnp.int32)
        for b in _BITS:
            cand = pos + b
            valid = cand <= NCAT
            gidx = jnp.minimum(cand, NCAT) - 1
            val = plsc.load_gather(u_v, [gidx])
            take = valid & (val <= q16)
            pos = jnp.where(take, cand, pos)
        return pos

    f_prev = search16(c_v[...])
    f_v[pl.ds(0, 16)] = jnp.where(quarter == 0, jnp.zeros((16,), jnp.int32), f_prev)

    @plsc.parallel_loop(0, NGRP, 1, unroll=16)
    def pass1(g):
        q16 = w_v[pl.ds(g * 16, 16)]
        jvec = (start + g * 16) + lane
        q16 = jnp.where(jvec == NCAT - 1, jnp.float32(2.0), q16)
        f_v[pl.ds(16 + g * 16, 16)] = search16(q16)

    @plsc.parallel_loop(0, NGRP, 1, unroll=4)
    def pass2(g):
        f_cur = f_v[pl.ds(16 + g * 16, 16)]
        f_pre = f_v[pl.ds(15 + g * 16, 16)]
        keep = f_cur > f_pre
        pv = p_v[pl.ds(g * 16, 16)]
        d = (1.0 - pv) - 1.0
        t = (-d) * (1.0 + d * (-0.5 + d * (jnp.float32(1.0 / 3.0) + d * -0.25)))
        powterm = jnp.exp(jnp.float32(-float(NCAT)) * t)
        denom = (1.0 - powterm) + jnp.float32(1e-12)
        p_v[pl.ds(g * 16, 16)] = jnp.where(keep, 1.0 / denom, jnp.float32(0.0))

    pltpu.sync_copy(p_v, out_hbm.at[row, pl.ds(start, CHUNK)])


_sap_call = functools.partial(
    pl.kernel,
    mesh=plsc.VectorSubcoreMesh(core_axis_name="c", subcore_axis_name="s"),
    out_type=jax.ShapeDtypeStruct((B, NCAT), jnp.float32),
    compiler_params=pltpu.CompilerParams(
        use_tc_tiling_on_sc=False, needs_layout_passes=False),
    scratch_types=[
        pltpu.VMEM((NCAT,), jnp.float32),
        pltpu.VMEM((CHUNK,), jnp.float32),
        pltpu.VMEM((CHUNK,), jnp.float32),
        pltpu.VMEM((CHUNK + 16,), jnp.int32),
        pltpu.VMEM((16,), jnp.float32),
        pltpu.VMEM((PADB,), jnp.float32),
        pltpu.VMEM((NTPR * PADB,), jnp.float32),
        pltpu.VMEM((4 * BLK,), jnp.float32),
        pltpu.VMEM((4 * BLK,), jnp.float32),
        pltpu.VMEM((16,), jnp.float32),
        pltpu.VMEM_SHARED((4, NTPR * PADB), jnp.float32),
        pltpu.VMEM_SHARED((4, NTPR, 16), jnp.float32),
    ],
)(_sap_body)


def kernel(inputs):
    b, c, h, w = inputs.shape
    ncat = c * h * w
    x2 = inputs.reshape(b, ncat)
    abs_in = jnp.abs(x2)
    s = jnp.sum(abs_in, axis=-1, keepdims=True)
    p = abs_in / s
    u_sorted = jnp.asarray(_U_SORTED)
    factor = _sap_call(p, u_sorted)
    return inputs * factor.reshape(b, c, h, w)
